# Initial kernel scaffold; baseline (speedup 1.0000x reference)
import jax
import jax.numpy as jnp
from jax.experimental import pallas as pl


def kernel(predictions, age, gender, age_means, age_stds, gender_adjustments, norm_weights):
    raise NotImplementedError("write your pallas kernel here")


# speedup vs baseline: 3.3118x; 3.3118x over previous
import functools

import jax
import jax.numpy as jnp
from jax import lax
from jax.experimental import pallas as pl
from jax.experimental.pallas import tpu as pltpu
from jax.experimental.pallas import tpu_sc as plsc

NUM_FACTORS = 128
AGE_BINS = 10
BATCH = 16384
AGE_MIN = 5.0
AGE_MAX = 21.0

NC = 2
NS = 16
NW = NC * NS
ROWS_PER_W = BATCH // NW
NCH = NUM_FACTORS // 16


def _body(pred_hbm, age_hbm, gen_hbm, means_hbm, stds_hbm, gadj_hbm, w_hbm,
          out_hbm,
          pred_v, age_v, gen_v, means_v, stds_v, gadj_v, w_v,
          a_v, c_v, bins_v, cidx_v, sem):
    wid = lax.axis_index("s") * NC + lax.axis_index("c")
    base = wid * ROWS_PER_W

    cp = pltpu.async_copy(pred_hbm.at[pl.ds(base, ROWS_PER_W), :], pred_v, sem)

    pltpu.sync_copy(age_hbm.at[pl.ds(base, ROWS_PER_W)], age_v)
    pltpu.sync_copy(gen_hbm.at[pl.ds(base, ROWS_PER_W)], gen_v)
    pltpu.sync_copy(means_hbm, means_v)
    pltpu.sync_copy(stds_hbm, stds_v)
    pltpu.sync_copy(gadj_hbm, gadj_v)
    pltpu.sync_copy(w_hbm, w_v)

    for j in range(NCH):
        sl = pl.ds(j * 16, 16)
        w = w_v[sl]
        gw0 = gadj_v[0, sl] * w
        gw1 = gadj_v[1, sl] * w
        for b in range(AGE_BINS):
            a = w / (stds_v[b, sl] + 1e-8)
            a_v[b, sl] = a
            ma = means_v[b, sl] * a
            c_v[2 * b, sl] = gw0 - ma
            c_v[2 * b + 1, sl] = gw1 - ma

    inv_range = 1.0 / (AGE_MAX - AGE_MIN)

    def bin_body(t, carry):
        sl = pl.ds(t * 16, 16)
        ages = age_v[sl].astype(jnp.float32)
        na = jnp.clip((ages - AGE_MIN) * inv_range, 0.0, 1.0)
        bins = (na * (AGE_BINS - 1)).astype(jnp.int32)
        bins_v[sl] = bins
        cidx_v[sl] = bins * 2 + gen_v[sl]
        return carry

    lax.fori_loop(0, ROWS_PER_W // 16, bin_body, 0)

    cp.wait()

    iota = lax.iota(jnp.int32, 16)

    def grp_body(t, carry):
        gsl = pl.ds(t * 16, 16)
        bins16 = bins_v[gsl]
        cidx16 = cidx_v[gsl]
        for i in range(16):
            r = t * 16 + i
            bs = lax.broadcast_in_dim(bins16[i], (16,), ())
            cs = lax.broadcast_in_dim(cidx16[i], (16,), ())
            for j in range(NCH):
                sl = pl.ds(j * 16, 16)
                col = iota + (j * 16)
                p = pred_v[r, sl]
                a = plsc.load_gather(a_v, [bs, col])
                c = plsc.load_gather(c_v, [cs, col])
                pred_v[r, sl] = p * a + c
        return carry

    lax.fori_loop(0, ROWS_PER_W // 16, grp_body, 0)

    pltpu.sync_copy(pred_v, out_hbm.at[pl.ds(base, ROWS_PER_W), :])


@functools.cache
def _build():
    mesh = plsc.VectorSubcoreMesh(
        core_axis_name="c", subcore_axis_name="s",
        num_cores=NC, num_subcores=NS)
    return pl.kernel(
        _body,
        out_type=jax.ShapeDtypeStruct((BATCH, NUM_FACTORS), jnp.float32),
        mesh=mesh,
        compiler_params=pltpu.CompilerParams(needs_layout_passes=False),
        scratch_types=[
            pltpu.VMEM((ROWS_PER_W, NUM_FACTORS), jnp.float32),
            pltpu.VMEM((ROWS_PER_W,), jnp.int32),
            pltpu.VMEM((ROWS_PER_W,), jnp.int32),
            pltpu.VMEM((AGE_BINS, NUM_FACTORS), jnp.float32),
            pltpu.VMEM((AGE_BINS, NUM_FACTORS), jnp.float32),
            pltpu.VMEM((2, NUM_FACTORS), jnp.float32),
            pltpu.VMEM((NUM_FACTORS,), jnp.float32),
            pltpu.VMEM((AGE_BINS, NUM_FACTORS), jnp.float32),
            pltpu.VMEM((2 * AGE_BINS, NUM_FACTORS), jnp.float32),
            pltpu.VMEM((ROWS_PER_W,), jnp.int32),
            pltpu.VMEM((ROWS_PER_W,), jnp.int32),
            pltpu.SemaphoreType.DMA,
        ],
    )


def kernel(predictions, age, gender, age_means, age_stds, gender_adjustments,
           norm_weights):
    age = age.astype(jnp.int32)
    gender = gender.astype(jnp.int32)
    return _build()(predictions, age, gender, age_means, age_stds,
                    gender_adjustments, norm_weights)


# speedup vs baseline: 4.8394x; 1.4613x over previous
import functools

import jax
import jax.numpy as jnp
from jax import lax
from jax.experimental import pallas as pl
from jax.experimental.pallas import tpu as pltpu
from jax.experimental.pallas import tpu_sc as plsc

NUM_FACTORS = 128
AGE_BINS = 10
BATCH = 16384
AGE_MIN = 5.0
AGE_MAX = 21.0

NC = 2
NS = 16
NW = NC * NS
ROWS_PER_W = BATCH // NW
NCH = NUM_FACTORS // 16


def _body(pred_hbm, age_hbm, gen_hbm, means_hbm, stds_hbm, gadj_hbm, w_hbm,
          out_hbm,
          pred_v, age_v, gen_v, means_v, stds_v, gadj_v, w_v,
          a_v, c_v, bins_v, cidx_v, sem):
    wid = lax.axis_index("s") * NC + lax.axis_index("c")
    base = wid * ROWS_PER_W

    cp = pltpu.async_copy(pred_hbm.at[pl.ds(base, ROWS_PER_W), :], pred_v, sem)

    pltpu.sync_copy(age_hbm.at[pl.ds(base, ROWS_PER_W)], age_v)
    pltpu.sync_copy(gen_hbm.at[pl.ds(base, ROWS_PER_W)], gen_v)
    pltpu.sync_copy(means_hbm, means_v)
    pltpu.sync_copy(stds_hbm, stds_v)
    pltpu.sync_copy(gadj_hbm, gadj_v)
    pltpu.sync_copy(w_hbm, w_v)

    for j in range(NCH):
        sl = pl.ds(j * 16, 16)
        w = w_v[sl]
        gw0 = gadj_v[0, sl] * w
        gw1 = gadj_v[1, sl] * w
        for b in range(AGE_BINS):
            a = w / (stds_v[b, sl] + 1e-8)
            a_v[b, sl] = a
            ma = means_v[b, sl] * a
            c_v[2 * b, sl] = gw0 - ma
            c_v[2 * b + 1, sl] = gw1 - ma

    inv_range = 1.0 / (AGE_MAX - AGE_MIN)

    @plsc.parallel_loop(0, ROWS_PER_W // 16)
    def bin_body(t):
        sl = pl.ds(t * 16, 16)
        ages = age_v[sl].astype(jnp.float32)
        na = jnp.clip((ages - AGE_MIN) * inv_range, 0.0, 1.0)
        bins = (na * (AGE_BINS - 1)).astype(jnp.int32)
        bins_v[sl] = bins
        cidx_v[sl] = bins * 2 + gen_v[sl]

    cp.wait()

    iota = lax.iota(jnp.int32, 16)

    @plsc.parallel_loop(0, ROWS_PER_W // 16)
    def grp_body(t):
        gsl = pl.ds(t * 16, 16)
        bins16 = bins_v[gsl]
        cidx16 = cidx_v[gsl]
        for i in range(16):
            r = t * 16 + i
            bs = lax.broadcast_in_dim(bins16[i], (16,), ())
            cs = lax.broadcast_in_dim(cidx16[i], (16,), ())
            ps = [pred_v[r, pl.ds(j * 16, 16)] for j in range(NCH)]
            avs = [plsc.load_gather(a_v, [bs, iota + (j * 16)])
                   for j in range(NCH)]
            cvs = [plsc.load_gather(c_v, [cs, iota + (j * 16)])
                   for j in range(NCH)]
            for j in range(NCH):
                pred_v[r, pl.ds(j * 16, 16)] = ps[j] * avs[j] + cvs[j]

    pltpu.sync_copy(pred_v, out_hbm.at[pl.ds(base, ROWS_PER_W), :])


@functools.cache
def _build():
    mesh = plsc.VectorSubcoreMesh(
        core_axis_name="c", subcore_axis_name="s",
        num_cores=NC, num_subcores=NS)
    return pl.kernel(
        _body,
        out_type=jax.ShapeDtypeStruct((BATCH, NUM_FACTORS), jnp.float32),
        mesh=mesh,
        compiler_params=pltpu.CompilerParams(needs_layout_passes=False),
        scratch_types=[
            pltpu.VMEM((ROWS_PER_W, NUM_FACTORS), jnp.float32),
            pltpu.VMEM((ROWS_PER_W,), jnp.int32),
            pltpu.VMEM((ROWS_PER_W,), jnp.int32),
            pltpu.VMEM((AGE_BINS, NUM_FACTORS), jnp.float32),
            pltpu.VMEM((AGE_BINS, NUM_FACTORS), jnp.float32),
            pltpu.VMEM((2, NUM_FACTORS), jnp.float32),
            pltpu.VMEM((NUM_FACTORS,), jnp.float32),
            pltpu.VMEM((AGE_BINS, NUM_FACTORS), jnp.float32),
            pltpu.VMEM((2 * AGE_BINS, NUM_FACTORS), jnp.float32),
            pltpu.VMEM((ROWS_PER_W,), jnp.int32),
            pltpu.VMEM((ROWS_PER_W,), jnp.int32),
            pltpu.SemaphoreType.DMA,
        ],
    )


def kernel(predictions, age, gender, age_means, age_stds, gender_adjustments,
           norm_weights):
    age = age.astype(jnp.int32)
    gender = gender.astype(jnp.int32)
    return _build()(predictions, age, gender, age_means, age_stds,
                    gender_adjustments, norm_weights)


# speedup vs baseline: 5.1225x; 1.0585x over previous
import functools

import jax
import jax.numpy as jnp
from jax import lax
from jax.experimental import pallas as pl
from jax.experimental.pallas import tpu as pltpu
from jax.experimental.pallas import tpu_sc as plsc

NUM_FACTORS = 128
AGE_BINS = 10
BATCH = 16384
AGE_MIN = 5.0
AGE_MAX = 21.0

NC = 2
NS = 16
NW = NC * NS
ROWS_PER_W = BATCH // NW
NCH = NUM_FACTORS // 16


def _body(pred_hbm, age_hbm, gen_hbm, means_hbm, stds_hbm, gadj_hbm, w_hbm,
          out_hbm,
          pred_v, age_v, gen_v, means_v, stds_v, gadj_v, w_v,
          a_v, c_v, bins_v, cidx_v, sem, sem_idx, sem_tab):
    wid = lax.axis_index("s") * NC + lax.axis_index("c")
    base = wid * ROWS_PER_W

    cp = pltpu.async_copy(pred_hbm.at[pl.ds(base, ROWS_PER_W), :], pred_v, sem)
    cp_age = pltpu.async_copy(age_hbm.at[pl.ds(base, ROWS_PER_W)], age_v, sem_idx)
    cp_gen = pltpu.async_copy(gen_hbm.at[pl.ds(base, ROWS_PER_W)], gen_v, sem_idx)
    cp_m = pltpu.async_copy(means_hbm, means_v, sem_tab)
    cp_s = pltpu.async_copy(stds_hbm, stds_v, sem_tab)
    cp_g = pltpu.async_copy(gadj_hbm, gadj_v, sem_tab)
    cp_w = pltpu.async_copy(w_hbm, w_v, sem_tab)
    cp_m.wait()
    cp_s.wait()
    cp_g.wait()
    cp_w.wait()

    @plsc.parallel_loop(0, AGE_BINS)
    def tab_body(b):
        ws = [w_v[pl.ds(j * 16, 16)] for j in range(NCH)]
        gw0s = [gadj_v[0, pl.ds(j * 16, 16)] * ws[j] for j in range(NCH)]
        gw1s = [gadj_v[1, pl.ds(j * 16, 16)] * ws[j] for j in range(NCH)]
        as_ = [ws[j] / (stds_v[b, pl.ds(j * 16, 16)] + 1e-8)
               for j in range(NCH)]
        mas = [means_v[b, pl.ds(j * 16, 16)] * as_[j] for j in range(NCH)]
        for j in range(NCH):
            sl = pl.ds(j * 16, 16)
            a_v[b, sl] = as_[j]
            c_v[2 * b, sl] = gw0s[j] - mas[j]
            c_v[2 * b + 1, sl] = gw1s[j] - mas[j]

    inv_range = 1.0 / (AGE_MAX - AGE_MIN)
    cp_age.wait()
    cp_gen.wait()

    @plsc.parallel_loop(0, ROWS_PER_W // 16)
    def bin_body(t):
        sl = pl.ds(t * 16, 16)
        ages = age_v[sl].astype(jnp.float32)
        na = jnp.clip((ages - AGE_MIN) * inv_range, 0.0, 1.0)
        bins = (na * (AGE_BINS - 1)).astype(jnp.int32)
        bins_v[sl] = bins
        cidx_v[sl] = bins * 2 + gen_v[sl]

    cp.wait()

    iota = lax.iota(jnp.int32, 16)

    GR = 8

    @plsc.parallel_loop(0, ROWS_PER_W // GR)
    def grp_body(t):
        gsl = pl.ds(t * GR, 16)
        bins16 = bins_v[gsl]
        cidx16 = cidx_v[gsl]
        for i in range(GR):
            r = t * GR + i
            bs = lax.broadcast_in_dim(bins16[i], (16,), ())
            cs = lax.broadcast_in_dim(cidx16[i], (16,), ())
            ps = [pred_v[r, pl.ds(j * 16, 16)] for j in range(NCH)]
            avs = [plsc.load_gather(a_v, [bs, iota + (j * 16)])
                   for j in range(NCH)]
            cvs = [plsc.load_gather(c_v, [cs, iota + (j * 16)])
                   for j in range(NCH)]
            for j in range(NCH):
                pred_v[r, pl.ds(j * 16, 16)] = ps[j] * avs[j] + cvs[j]

    pltpu.sync_copy(pred_v, out_hbm.at[pl.ds(base, ROWS_PER_W), :])


@functools.cache
def _build():
    mesh = plsc.VectorSubcoreMesh(
        core_axis_name="c", subcore_axis_name="s",
        num_cores=NC, num_subcores=NS)
    return pl.kernel(
        _body,
        out_type=jax.ShapeDtypeStruct((BATCH, NUM_FACTORS), jnp.float32),
        mesh=mesh,
        compiler_params=pltpu.CompilerParams(needs_layout_passes=False),
        scratch_types=[
            pltpu.VMEM((ROWS_PER_W, NUM_FACTORS), jnp.float32),
            pltpu.VMEM((ROWS_PER_W,), jnp.int32),
            pltpu.VMEM((ROWS_PER_W,), jnp.int32),
            pltpu.VMEM((AGE_BINS, NUM_FACTORS), jnp.float32),
            pltpu.VMEM((AGE_BINS, NUM_FACTORS), jnp.float32),
            pltpu.VMEM((2, NUM_FACTORS), jnp.float32),
            pltpu.VMEM((NUM_FACTORS,), jnp.float32),
            pltpu.VMEM((AGE_BINS, NUM_FACTORS), jnp.float32),
            pltpu.VMEM((2 * AGE_BINS, NUM_FACTORS), jnp.float32),
            pltpu.VMEM((ROWS_PER_W + 16,), jnp.int32),
            pltpu.VMEM((ROWS_PER_W + 16,), jnp.int32),
            pltpu.SemaphoreType.DMA,
            pltpu.SemaphoreType.DMA,
            pltpu.SemaphoreType.DMA,
        ],
    )


def kernel(predictions, age, gender, age_means, age_stds, gender_adjustments,
           norm_weights):
    age = age.astype(jnp.int32)
    gender = gender.astype(jnp.int32)
    return _build()(predictions, age, gender, age_means, age_stds,
                    gender_adjustments, norm_weights)
